# baseline (speedup 1.0000x reference)
import functools

import jax
import jax.numpy as jnp
from jax import lax
from jax.experimental import pallas as pl
from jax.experimental.pallas import tpu as pltpu
from jax.experimental.pallas import tpu_sc as plsc

N = 10000
E = 320000
G = 16
D = 512
L = 8
NT = 51
GT = 11
HID = 32

BN = 1000
GRID_N = N // BN


def _l2norm_rows(y):
    n = jnp.sqrt(jnp.sum(y * y, axis=1, keepdims=True))
    return y / jnp.maximum(n, 1e-12)


def _premp_body(x_ref, w_ref, o_ref):
    y = jax.nn.relu(jnp.dot(x_ref[...], w_ref[...],
                            preferred_element_type=jnp.float32))
    o_ref[...] = _l2norm_rows(y)


def _premp(x128, w128):
    return pl.pallas_call(
        _premp_body,
        grid=(GRID_N,),
        in_specs=[pl.BlockSpec((BN, 128), lambda i: (i, 0)),
                  pl.BlockSpec((128, D), lambda i: (0, 0))],
        out_specs=pl.BlockSpec((BN, D), lambda i: (i, 0)),
        out_shape=jax.ShapeDtypeStruct((N, D), jnp.float32),
    )(x128, w128)


def _proj_body(h_ref, w_ref, o_ref):
    o_ref[...] = jnp.dot(h_ref[...], w_ref[...],
                         preferred_element_type=jnp.float32)


def _proj(h, wcat):
    return pl.pallas_call(
        _proj_body,
        grid=(GRID_N,),
        in_specs=[pl.BlockSpec((BN, D), lambda i: (i, 0)),
                  pl.BlockSpec((D, 4 * D), lambda i: (0, 0))],
        out_specs=pl.BlockSpec((BN, 4 * D), lambda i: (i, 0)),
        out_shape=jax.ShapeDtypeStruct((N, 4 * D), jnp.float32),
    )(h, wcat)


def _epi_body(agg_ref, s_ref, hin_ref, o_ref):
    y = jax.nn.relu(agg_ref[...] + s_ref[...])
    o_ref[...] = _l2norm_rows(y) + hin_ref[...]


def _epilogue(agg, s, h_in):
    return pl.pallas_call(
        _epi_body,
        grid=(GRID_N,),
        in_specs=[pl.BlockSpec((BN, D), lambda i: (i, 0))] * 3,
        out_specs=pl.BlockSpec((BN, D), lambda i: (i, 0)),
        out_shape=jax.ShapeDtypeStruct((N, D), jnp.float32),
    )(agg, s, h_in)


def _heads_body(h_ref, w1_ref, w2b_ref, gmask_ref, b2_ref, batch_ref,
                np_ref, pool_ref):
    i = pl.program_id(0)
    hb = _l2norm_rows(h_ref[...])
    z = jax.nn.relu(jnp.dot(hb, w1_ref[...], preferred_element_type=jnp.float32))
    ss = jnp.dot(z * z, gmask_ref[...], preferred_element_type=jnp.float32)
    denom = jnp.maximum(jnp.sqrt(ss), 1e-12)
    num = jnp.dot(z, w2b_ref[...], preferred_element_type=jnp.float32)
    np_ref[...] = num / denom + b2_ref[...]
    mask = (batch_ref[0] == jax.lax.broadcasted_iota(jnp.int32, (G, BN), 0)
            ).astype(jnp.float32)
    pool = jnp.dot(mask, hb, preferred_element_type=jnp.float32)

    @pl.when(i == 0)
    def _():
        pool_ref[...] = jnp.zeros_like(pool_ref)

    pool_ref[...] += pool


def _heads(h, w1r, w2b, gmask, b2row, batch3):
    return pl.pallas_call(
        _heads_body,
        grid=(GRID_N,),
        in_specs=[pl.BlockSpec((BN, D), lambda i: (i, 0)),
                  pl.BlockSpec((D, NT * HID), lambda i: (0, 0)),
                  pl.BlockSpec((NT * HID, NT), lambda i: (0, 0)),
                  pl.BlockSpec((NT * HID, NT), lambda i: (0, 0)),
                  pl.BlockSpec((1, NT), lambda i: (0, 0)),
                  pl.BlockSpec((1, 1, BN), lambda i: (i, 0, 0))],
        out_specs=[pl.BlockSpec((BN, NT), lambda i: (i, 0)),
                   pl.BlockSpec((G, D), lambda i: (0, 0))],
        out_shape=[jax.ShapeDtypeStruct((N, NT), jnp.float32),
                   jax.ShapeDtypeStruct((G, D), jnp.float32)],
    )(h, w1r, w2b, gmask, b2row, batch3)


def _ghead_body(g_ref, wg1_ref, wg2_ref, bg2_ref, o_ref):
    gh = _l2norm_rows(jax.nn.relu(
        jnp.dot(g_ref[...], wg1_ref[...], preferred_element_type=jnp.float32)))
    o_ref[...] = jnp.dot(gh, wg2_ref[...],
                         preferred_element_type=jnp.float32) + bg2_ref[...]


def _ghead(g, wg1, wg2, bg2row):
    return pl.pallas_call(
        _ghead_body,
        in_specs=[pl.BlockSpec((G, D), lambda: (0, 0)),
                  pl.BlockSpec((D, D), lambda: (0, 0)),
                  pl.BlockSpec((D, GT), lambda: (0, 0)),
                  pl.BlockSpec((1, GT), lambda: (0, 0))],
        out_specs=pl.BlockSpec((G, GT), lambda: (0, 0)),
        out_shape=jax.ShapeDtypeStruct((G, GT), jnp.float32),
    )(g, wg1, wg2, bg2row)


NW = 32
NPW = 320
NPAD = NW * NPW
PN = 80
NPASS = NPW // PN
EC = 16
NSL = D // 16
OFFS_LEN = NPAD + 88

_sc_mesh = plsc.VectorSubcoreMesh(core_axis_name="c", subcore_axis_name="s")


def _splat16(vec, j):
    idx = jnp.full((16,), j, dtype=jnp.int32)
    return vec.at[idx].get(mode="promise_in_bounds")


@functools.partial(
    pl.kernel,
    out_type=jax.ShapeDtypeStruct((NPAD * D,), jnp.float32),
    mesh=_sc_mesh,
    scratch_types=[
        pltpu.VMEM((PN * D,), jnp.float32),
        pltpu.VMEM((PN * D,), jnp.float32),
        pltpu.VMEM((EC, D), jnp.float32),
        pltpu.VMEM((EC, D), jnp.float32),
        pltpu.VMEM((EC,), jnp.int32),
        pltpu.VMEM((EC,), jnp.int32),
        pltpu.VMEM((336,), jnp.int32),
        pltpu.SemaphoreType.DMA,
    ],
    compiler_params=pltpu.CompilerParams(needs_layout_passes=False),
)
def _edge_sc(k_hbm, q_hbm, v_hbm, src_hbm, dst_hbm, offs_hbm, agg_hbm,
             acc, kbuf, qbuf, vbuf, srcbuf, dstbuf, offsbuf, sem):
    wid = lax.axis_index("s") * 2 + lax.axis_index("c")
    wbase = pl.multiple_of(wid * NPW, 16)
    pltpu.sync_copy(offs_hbm.at[pl.ds(wbase, 336)], offsbuf)

    lanes = lax.iota(jnp.int32, 16)
    zero16 = jnp.zeros((16,), jnp.float32)

    def pass_body(p, pcarry):
        nbase = wbase + pl.multiple_of(p * PN, 16)
        e0 = offsbuf[pl.ds(pl.multiple_of(p * PN, 16), 16)][0]
        e1 = offsbuf[pl.ds(pl.multiple_of(p * PN + PN, 16), 16)][0]

        def zrow(r, carry):
            rb = pl.multiple_of(r * D, 16)
            for sl in range(NSL):
                acc[pl.ds(rb + sl * 16, 16)] = zero16
            return carry

        lax.fori_loop(0, PN, zrow, 0)

        kb = pl.multiple_of(nbase * D, 16)
        pltpu.sync_copy(k_hbm.at[pl.ds(kb, PN * D)], kbuf)

        bstart = e0 & ~15
        nchunks = (e1 - bstart + 15) >> 4

        def chunk_body(ci, carry):
            b = pl.multiple_of(bstart + ci * EC, 16)
            pltpu.sync_copy(src_hbm.at[pl.ds(b, EC)], srcbuf)
            pltpu.sync_copy(dst_hbm.at[pl.ds(b, EC)], dstbuf)
            cq = pltpu.async_copy(q_hbm.at[srcbuf], qbuf, sem)
            cv = pltpu.async_copy(v_hbm.at[srcbuf], vbuf, sem)
            cq.wait()
            cv.wait()
            dlv = dstbuf[...] - nbase

            def edge_body(j, ecarry):
                dlb = _splat16(dlv, j)
                valid = (dlb >= 0) & (dlb < PN)
                rowbase = jnp.clip(dlb, 0, PN - 1) * D + lanes
                for sl in range(NSL):
                    ds = pl.ds(sl * 16, 16)
                    idxv = rowbase + (sl * 16)
                    kv = plsc.load_gather(kbuf, [idxv])
                    qv = qbuf[j, ds]
                    vv = vbuf[j, ds]
                    t = jnp.exp(-(kv + qv))
                    m = vv / (1.0 + t)
                    plsc.addupdate_scatter(acc, [idxv], m, mask=valid)
                return ecarry

            lax.fori_loop(0, EC, edge_body, 0)
            return carry

        lax.fori_loop(0, nchunks, chunk_body, 0)
        pltpu.sync_copy(acc, agg_hbm.at[pl.ds(kb, PN * D)])
        return pcarry

    lax.fori_loop(0, NPASS, pass_body, 0)


def _edge_prep(src, dst):
    perm = jnp.argsort(dst)
    src_s = jnp.pad(src[perm].astype(jnp.int32), (0, EC))
    dst_s = jnp.pad(dst[perm].astype(jnp.int32), (0, EC),
                    constant_values=NPAD)
    offs = jnp.searchsorted(
        dst_s[:E], jnp.arange(OFFS_LEN, dtype=jnp.int32)).astype(jnp.int32)
    return src_s, dst_s, offs


def _edge_stage(k, q, v, src_s, dst_s, offs):
    k_pad = jnp.pad(k, ((0, NPAD - N), (0, 0))).reshape(-1)
    agg = _edge_sc(k_pad, q, v, src_s, dst_s, offs)
    return agg.reshape(NPAD, D)[:N]


def kernel(x, edge_index, batch, y, y_graph, Wpre, Wk, Wq, Wv, Ws,
           W1, W2, b2, Wg1, Wg2, bg2):
    src = edge_index[0]
    dst = edge_index[1]
    src_s, dst_s, offs = _edge_prep(src, dst)

    x128 = jnp.pad(x, ((0, 0), (0, 128 - x.shape[1])))
    w128 = jnp.pad(Wpre, ((0, 128 - Wpre.shape[0]), (0, 0)))
    h = _premp(x128, w128)

    wcat = jnp.concatenate([Wk, Wq, Wv, Ws], axis=2)

    for l in range(L):
        h_in = h
        kqvs = _proj(h, wcat[l])
        k = kqvs[:, 0:D]
        q = kqvs[:, D:2 * D]
        v = kqvs[:, 2 * D:3 * D]
        s = kqvs[:, 3 * D:4 * D]
        agg = _edge_stage(k, q, v, src_s, dst_s, offs)
        h = _epilogue(agg, s, h_in)

    w1r = W1.transpose(1, 0, 2).reshape(D, NT * HID)
    hh = jnp.arange(NT)
    kk = jnp.arange(HID)
    w2b = jnp.zeros((NT, HID, NT), jnp.float32).at[
        hh[:, None], kk[None, :], hh[:, None]].set(W2).reshape(NT * HID, NT)
    gmask = jnp.zeros((NT, HID, NT), jnp.float32).at[
        hh[:, None], kk[None, :], hh[:, None]].set(1.0).reshape(NT * HID, NT)
    batch3 = batch.astype(jnp.int32).reshape(GRID_N, 1, BN)
    node_pred, g = _heads(h, w1r, w2b, gmask, b2[None, :], batch3)
    graph_pred = _ghead(g, Wg1, Wg2, bg2[None, :])

    pred = jnp.vstack([jnp.pad(node_pred, ((0, 0), (0, GT))),
                       jnp.pad(graph_pred, ((0, 0), (NT, 0)))])
    true = jnp.vstack([jnp.pad(y, ((0, 0), (0, GT))),
                       jnp.pad(y_graph, ((0, 0), (NT, 0)))])
    return pred, true


# speedup vs baseline: 1.1530x; 1.1530x over previous
import functools

import jax
import jax.numpy as jnp
from jax import lax
from jax.experimental import pallas as pl
from jax.experimental.pallas import tpu as pltpu
from jax.experimental.pallas import tpu_sc as plsc

N = 10000
E = 320000
G = 16
D = 512
L = 8
NT = 51
GT = 11
HID = 32

BN = 1000
GRID_N = N // BN


def _l2norm_rows(y):
    n = jnp.sqrt(jnp.sum(y * y, axis=1, keepdims=True))
    return y / jnp.maximum(n, 1e-12)


def _premp_body(x_ref, w_ref, o_ref):
    y = jax.nn.relu(jnp.dot(x_ref[...], w_ref[...],
                            preferred_element_type=jnp.float32))
    o_ref[...] = _l2norm_rows(y)


def _premp(x128, w128):
    return pl.pallas_call(
        _premp_body,
        grid=(GRID_N,),
        in_specs=[pl.BlockSpec((BN, 128), lambda i: (i, 0)),
                  pl.BlockSpec((128, D), lambda i: (0, 0))],
        out_specs=pl.BlockSpec((BN, D), lambda i: (i, 0)),
        out_shape=jax.ShapeDtypeStruct((N, D), jnp.float32),
    )(x128, w128)


def _proj_body(h_ref, w_ref, o_ref):
    o_ref[...] = jnp.dot(h_ref[...], w_ref[...],
                         preferred_element_type=jnp.float32)


def _proj(h, wcat):
    return pl.pallas_call(
        _proj_body,
        grid=(GRID_N,),
        in_specs=[pl.BlockSpec((BN, D), lambda i: (i, 0)),
                  pl.BlockSpec((D, 4 * D), lambda i: (0, 0))],
        out_specs=pl.BlockSpec((BN, 4 * D), lambda i: (i, 0)),
        out_shape=jax.ShapeDtypeStruct((N, 4 * D), jnp.float32),
    )(h, wcat)


def _epi_body(agg_ref, s_ref, hin_ref, o_ref):
    y = jax.nn.relu(agg_ref[...] + s_ref[...])
    o_ref[...] = _l2norm_rows(y) + hin_ref[...]


def _epilogue(agg, s, h_in):
    return pl.pallas_call(
        _epi_body,
        grid=(GRID_N,),
        in_specs=[pl.BlockSpec((BN, D), lambda i: (i, 0))] * 3,
        out_specs=pl.BlockSpec((BN, D), lambda i: (i, 0)),
        out_shape=jax.ShapeDtypeStruct((N, D), jnp.float32),
    )(agg, s, h_in)


def _heads_body(h_ref, w1_ref, w2b_ref, gmask_ref, b2_ref, batch_ref,
                np_ref, pool_ref):
    i = pl.program_id(0)
    hb = _l2norm_rows(h_ref[...])
    z = jax.nn.relu(jnp.dot(hb, w1_ref[...], preferred_element_type=jnp.float32))
    ss = jnp.dot(z * z, gmask_ref[...], preferred_element_type=jnp.float32)
    denom = jnp.maximum(jnp.sqrt(ss), 1e-12)
    num = jnp.dot(z, w2b_ref[...], preferred_element_type=jnp.float32)
    np_ref[...] = num / denom + b2_ref[...]
    mask = (batch_ref[0] == jax.lax.broadcasted_iota(jnp.int32, (G, BN), 0)
            ).astype(jnp.float32)
    pool = jnp.dot(mask, hb, preferred_element_type=jnp.float32)

    @pl.when(i == 0)
    def _():
        pool_ref[...] = jnp.zeros_like(pool_ref)

    pool_ref[...] += pool


def _heads(h, w1r, w2b, gmask, b2row, batch3):
    return pl.pallas_call(
        _heads_body,
        grid=(GRID_N,),
        in_specs=[pl.BlockSpec((BN, D), lambda i: (i, 0)),
                  pl.BlockSpec((D, NT * HID), lambda i: (0, 0)),
                  pl.BlockSpec((NT * HID, NT), lambda i: (0, 0)),
                  pl.BlockSpec((NT * HID, NT), lambda i: (0, 0)),
                  pl.BlockSpec((1, NT), lambda i: (0, 0)),
                  pl.BlockSpec((1, 1, BN), lambda i: (i, 0, 0))],
        out_specs=[pl.BlockSpec((BN, NT), lambda i: (i, 0)),
                   pl.BlockSpec((G, D), lambda i: (0, 0))],
        out_shape=[jax.ShapeDtypeStruct((N, NT), jnp.float32),
                   jax.ShapeDtypeStruct((G, D), jnp.float32)],
    )(h, w1r, w2b, gmask, b2row, batch3)


def _ghead_body(g_ref, wg1_ref, wg2_ref, bg2_ref, o_ref):
    gh = _l2norm_rows(jax.nn.relu(
        jnp.dot(g_ref[...], wg1_ref[...], preferred_element_type=jnp.float32)))
    o_ref[...] = jnp.dot(gh, wg2_ref[...],
                         preferred_element_type=jnp.float32) + bg2_ref[...]


def _ghead(g, wg1, wg2, bg2row):
    return pl.pallas_call(
        _ghead_body,
        in_specs=[pl.BlockSpec((G, D), lambda: (0, 0)),
                  pl.BlockSpec((D, D), lambda: (0, 0)),
                  pl.BlockSpec((D, GT), lambda: (0, 0)),
                  pl.BlockSpec((1, GT), lambda: (0, 0))],
        out_specs=pl.BlockSpec((G, GT), lambda: (0, 0)),
        out_shape=jax.ShapeDtypeStruct((G, GT), jnp.float32),
    )(g, wg1, wg2, bg2row)


NW = 32
NPW = 320
NPAD = NW * NPW
PN = 40
NPASS = NPW // PN
EC = 32
NSL = D // 16
OFFS_LEN = NPAD + 88

_sc_mesh = plsc.VectorSubcoreMesh(core_axis_name="c", subcore_axis_name="s")


def _splat16(vec, j):
    idx = jnp.full((16,), j, dtype=jnp.int32)
    return vec.at[idx].get(mode="promise_in_bounds")


@functools.partial(
    pl.kernel,
    out_type=jax.ShapeDtypeStruct((NPAD * D,), jnp.float32),
    mesh=_sc_mesh,
    scratch_types=[
        pltpu.VMEM((PN * D,), jnp.float32),
        pltpu.VMEM((PN * D,), jnp.float32),
        pltpu.VMEM((2, EC, D), jnp.float32),
        pltpu.VMEM((2, EC, D), jnp.float32),
        pltpu.VMEM((2, EC), jnp.int32),
        pltpu.VMEM((2, EC), jnp.int32),
        pltpu.VMEM((336,), jnp.int32),
        pltpu.SemaphoreType.DMA,
        pltpu.SemaphoreType.DMA,
    ],
    compiler_params=pltpu.CompilerParams(needs_layout_passes=False),
)
def _edge_sc(k_hbm, q_hbm, v_hbm, src_hbm, dst_hbm, offs_hbm, agg_hbm,
             acc, kbuf, qbuf, vbuf, srcbuf, dstbuf, offsbuf, sem_idx, sem_g):
    wid = lax.axis_index("s") * 2 + lax.axis_index("c")
    wbase = pl.multiple_of(wid * NPW, 16)
    pltpu.sync_copy(offs_hbm.at[pl.ds(wbase, 336)], offsbuf)

    lanes = lax.iota(jnp.int32, 16)
    zero16 = jnp.zeros((16,), jnp.float32)

    def idx_start(b, slot):
        b = pl.multiple_of(b, 8)
        pltpu.async_copy(src_hbm.at[pl.ds(b, EC)], srcbuf.at[slot], sem_idx)
        pltpu.async_copy(dst_hbm.at[pl.ds(b, EC)], dstbuf.at[slot], sem_idx)

    def idx_wait(b, slot):
        b = pl.multiple_of(b, 8)
        pltpu.make_async_copy(
            src_hbm.at[pl.ds(b, EC)], srcbuf.at[slot], sem_idx).wait()
        pltpu.make_async_copy(
            dst_hbm.at[pl.ds(b, EC)], dstbuf.at[slot], sem_idx).wait()

    def gather_start(slot):
        pltpu.async_copy(q_hbm.at[srcbuf.at[slot]], qbuf.at[slot], sem_g)
        pltpu.async_copy(v_hbm.at[srcbuf.at[slot]], vbuf.at[slot], sem_g)

    def gather_wait(slot):
        pltpu.make_async_copy(
            q_hbm.at[srcbuf.at[slot]], qbuf.at[slot], sem_g).wait()
        pltpu.make_async_copy(
            v_hbm.at[srcbuf.at[slot]], vbuf.at[slot], sem_g).wait()

    def pass_body(p, pcarry):
        poff = pl.multiple_of(p * PN, 8)
        nbase = wbase + poff
        e0 = offsbuf[pl.ds(poff, 16)][0]
        e1 = offsbuf[pl.ds(pl.multiple_of(poff + PN, 8), 16)][0]

        def zrow(r, carry):
            rb = pl.multiple_of(r * D, 16)
            for sl in range(NSL):
                acc[pl.ds(rb + sl * 16, 16)] = zero16
            return carry

        lax.fori_loop(0, PN, zrow, 0)

        kb = pl.multiple_of(nbase * D, 16)
        pltpu.sync_copy(k_hbm.at[pl.ds(kb, PN * D)], kbuf)

        bstart = e0 & ~(EC - 1)
        nchunks = (e1 - bstart + (EC - 1)) >> 5

        @pl.when(nchunks > 0)
        def _prologue():
            idx_start(bstart, 0)
            idx_wait(bstart, 0)
            gather_start(0)

            @pl.when(nchunks > 1)
            def _():
                idx_start(bstart + EC, 1)

        def half_body(ci, slot):
            nslot = 1 - slot
            b1 = pl.multiple_of(bstart + (ci + 1) * EC, 8)
            b2 = pl.multiple_of(bstart + (ci + 2) * EC, 8)

            @pl.when(ci + 1 < nchunks)
            def _():
                idx_wait(b1, nslot)
                gather_start(nslot)

            gather_wait(slot)
            dlv0 = dstbuf[slot, pl.ds(0, 16)] - nbase
            dlv1 = dstbuf[slot, pl.ds(16, 16)] - nbase

            @pl.when(ci + 2 < nchunks)
            def _():
                idx_start(b2, slot)

            def edge_body(j, ecarry):
                dlb = _splat16(jnp.where(j < 16, dlv0, dlv1), j & 15)
                valid = (dlb >= 0) & (dlb < PN)
                rowbase = jnp.clip(dlb, 0, PN - 1) * D + lanes
                for sl in range(NSL):
                    ds = pl.ds(sl * 16, 16)
                    idxv = rowbase + (sl * 16)
                    kv = plsc.load_gather(kbuf, [idxv])
                    qv = qbuf[slot, j, ds]
                    vv = vbuf[slot, j, ds]
                    t = jnp.exp(-(kv + qv))
                    m = vv / (1.0 + t)
                    plsc.addupdate_scatter(acc, [idxv], m, mask=valid)
                return ecarry

            lax.fori_loop(0, EC, edge_body, 0)

        def chunk_pair(cb, carry):
            ci = cb * 2
            half_body(ci, 0)

            @pl.when(ci + 1 < nchunks)
            def _():
                half_body(ci + 1, 1)

            return carry

        lax.fori_loop(0, (nchunks + 1) >> 1, chunk_pair, 0)
        pltpu.sync_copy(acc, agg_hbm.at[pl.ds(kb, PN * D)])
        return pcarry

    lax.fori_loop(0, NPASS, pass_body, 0)


def _edge_prep(src, dst):
    perm = jnp.argsort(dst)
    src_s = jnp.pad(src[perm].astype(jnp.int32), (0, EC))
    dst_s = jnp.pad(dst[perm].astype(jnp.int32), (0, EC),
                    constant_values=NPAD)
    offs = jnp.searchsorted(
        dst_s[:E], jnp.arange(OFFS_LEN, dtype=jnp.int32)).astype(jnp.int32)
    return src_s, dst_s, offs


def _edge_stage(k, q, v, src_s, dst_s, offs):
    k_pad = jnp.pad(k, ((0, NPAD - N), (0, 0))).reshape(-1)
    agg = _edge_sc(k_pad, q, v, src_s, dst_s, offs)
    return agg.reshape(NPAD, D)[:N]


def kernel(x, edge_index, batch, y, y_graph, Wpre, Wk, Wq, Wv, Ws,
           W1, W2, b2, Wg1, Wg2, bg2):
    src = edge_index[0]
    dst = edge_index[1]
    src_s, dst_s, offs = _edge_prep(src, dst)

    x128 = jnp.pad(x, ((0, 0), (0, 128 - x.shape[1])))
    w128 = jnp.pad(Wpre, ((0, 128 - Wpre.shape[0]), (0, 0)))
    h = _premp(x128, w128)

    wcat = jnp.concatenate([Wk, Wq, Wv, Ws], axis=2)

    for l in range(L):
        h_in = h
        kqvs = _proj(h, wcat[l])
        k = kqvs[:, 0:D]
        q = kqvs[:, D:2 * D]
        v = kqvs[:, 2 * D:3 * D]
        s = kqvs[:, 3 * D:4 * D]
        agg = _edge_stage(k, q, v, src_s, dst_s, offs)
        h = _epilogue(agg, s, h_in)

    w1r = W1.transpose(1, 0, 2).reshape(D, NT * HID)
    hh = jnp.arange(NT)
    kk = jnp.arange(HID)
    w2b = jnp.zeros((NT, HID, NT), jnp.float32).at[
        hh[:, None], kk[None, :], hh[:, None]].set(W2).reshape(NT * HID, NT)
    gmask = jnp.zeros((NT, HID, NT), jnp.float32).at[
        hh[:, None], kk[None, :], hh[:, None]].set(1.0).reshape(NT * HID, NT)
    batch3 = batch.astype(jnp.int32).reshape(GRID_N, 1, BN)
    node_pred, g = _heads(h, w1r, w2b, gmask, b2[None, :], batch3)
    graph_pred = _ghead(g, Wg1, Wg2, bg2[None, :])

    pred = jnp.vstack([jnp.pad(node_pred, ((0, 0), (0, GT))),
                       jnp.pad(graph_pred, ((0, 0), (NT, 0)))])
    true = jnp.vstack([jnp.pad(y, ((0, 0), (0, GT))),
                       jnp.pad(y_graph, ((0, 0), (NT, 0)))])
    return pred, true
